# baseline (speedup 1.0000x reference)
from functools import partial

import jax
import jax.numpy as jnp
from jax.experimental import pallas as pl
from jax.experimental.pallas import tpu as pltpu

_EPS = 1e-5
_VMEM_LIMIT = 14 * 1024 * 1024


def _fused_body(x_ref, gamma_ref, beta_ref, w_ref, b_ref, o_ref, *,
                inv_n, gsize):
    x = x_ref[0].astype(jnp.float32)
    C = x.shape[0]

    s1 = jnp.sum(x, axis=-1, keepdims=True)
    s2 = jnp.sum(x * x, axis=-1, keepdims=True)

    row = jax.lax.broadcasted_iota(jnp.int32, (C, C), 0) // gsize
    col = jax.lax.broadcasted_iota(jnp.int32, (C, C), 1) // gsize
    mask = (row == col).astype(jnp.float32)
    s12 = jnp.concatenate([s1, s2], axis=1)
    gs = jnp.dot(mask, s12, preferred_element_type=jnp.float32,
                 precision=jax.lax.Precision.HIGHEST)

    mean = gs[:, 0:1] * inv_n
    ex2 = gs[:, 1:2] * inv_n
    rstd = jax.lax.rsqrt(ex2 - mean * mean + _EPS)
    scale = gamma_ref[...] * rstd
    shift = beta_ref[...] - mean * scale

    y = (x * scale + shift).astype(jnp.bfloat16)
    out = jnp.dot(w_ref[...], y, preferred_element_type=jnp.float32)
    o_ref[0] = (out + b_ref[...]).astype(o_ref.dtype)


def kernel(x, gamma, beta, w, b):
    B, C, H, W = x.shape
    HW = H * W
    num_groups = C // 4 if C % 4 == 0 else C
    gsize = C // num_groups
    inv_n = 1.0 / float(gsize * HW)

    xf = x.astype(jnp.bfloat16).reshape(B, C, HW)
    gamma2 = jnp.asarray(gamma, jnp.float32).reshape(C, 1)
    beta2 = jnp.asarray(beta, jnp.float32).reshape(C, 1)
    b2 = jnp.asarray(b, jnp.float32).reshape(C, 1)
    wbf = jnp.asarray(w).astype(jnp.bfloat16)

    out = pl.pallas_call(
        partial(_fused_body, inv_n=inv_n, gsize=gsize),
        out_shape=jax.ShapeDtypeStruct((B, C, HW), jnp.bfloat16),
        grid=(B,),
        in_specs=[
            pl.BlockSpec((1, C, HW), lambda bb: (bb, 0, 0)),
            pl.BlockSpec((C, 1), lambda bb: (0, 0)),
            pl.BlockSpec((C, 1), lambda bb: (0, 0)),
            pl.BlockSpec((C, C), lambda bb: (0, 0)),
            pl.BlockSpec((C, 1), lambda bb: (0, 0)),
        ],
        out_specs=pl.BlockSpec((1, C, HW), lambda bb: (bb, 0, 0)),
        compiler_params=pltpu.CompilerParams(
            dimension_semantics=("parallel",),
            vmem_limit_bytes=_VMEM_LIMIT),
    )(xf, gamma2, beta2, wbf, b2)

    return out.reshape(B, C, H, W).astype(x.dtype)


# speedup vs baseline: 1.0878x; 1.0878x over previous
from functools import partial

import jax
import jax.numpy as jnp
from jax.experimental import pallas as pl
from jax.experimental.pallas import tpu as pltpu

_EPS = 1e-5
_VMEM_LIMIT = 112 * 1024 * 1024


def _fused_body(x_ref, gamma_ref, beta_ref, w_ref, b_ref, o_ref, *,
                inv_n, gsize):
    x = x_ref[0]
    C = x.shape[0]

    s1 = jnp.sum(x, axis=-1, keepdims=True)
    s2 = jnp.sum(x * x, axis=-1, keepdims=True)

    row = jax.lax.broadcasted_iota(jnp.int32, (C, C), 0) // gsize
    col = jax.lax.broadcasted_iota(jnp.int32, (C, C), 1) // gsize
    mask = (row == col).astype(jnp.float32)
    s12 = jnp.concatenate([s1, s2], axis=1)
    gs = jnp.dot(mask, s12, preferred_element_type=jnp.float32,
                 precision=jax.lax.Precision.HIGHEST)

    mean = gs[:, 0:1] * inv_n
    ex2 = gs[:, 1:2] * inv_n
    rstd = jax.lax.rsqrt(ex2 - mean * mean + _EPS)
    scale = gamma_ref[...] * rstd
    shift = beta_ref[...] - mean * scale

    y = (x * scale + shift).astype(jnp.bfloat16)
    out = jnp.dot(w_ref[...], y, preferred_element_type=jnp.float32)
    o_ref[0] = (out + b_ref[...]).astype(o_ref.dtype)


def kernel(x, gamma, beta, w, b):
    B, C, H, W = x.shape
    HW = H * W
    num_groups = C // 4 if C % 4 == 0 else C
    gsize = C // num_groups
    inv_n = 1.0 / float(gsize * HW)

    xf = x.reshape(B, C, HW)
    gamma2 = jnp.asarray(gamma, jnp.float32).reshape(C, 1)
    beta2 = jnp.asarray(beta, jnp.float32).reshape(C, 1)
    b2 = jnp.asarray(b, jnp.float32).reshape(C, 1)
    wbf = jnp.asarray(w).astype(jnp.bfloat16)

    out = pl.pallas_call(
        partial(_fused_body, inv_n=inv_n, gsize=gsize),
        out_shape=jax.ShapeDtypeStruct((B, C, HW), jnp.bfloat16),
        grid=(B,),
        in_specs=[
            pl.BlockSpec((1, C, HW), lambda bb: (bb, 0, 0)),
            pl.BlockSpec((C, 1), lambda bb: (0, 0)),
            pl.BlockSpec((C, 1), lambda bb: (0, 0)),
            pl.BlockSpec((C, C), lambda bb: (0, 0)),
            pl.BlockSpec((C, 1), lambda bb: (0, 0)),
        ],
        out_specs=pl.BlockSpec((1, C, HW), lambda bb: (bb, 0, 0)),
        compiler_params=pltpu.CompilerParams(
            dimension_semantics=("parallel",),
            vmem_limit_bytes=_VMEM_LIMIT),
    )(xf, gamma2, beta2, wbf, b2)

    return out.reshape(B, C, H, W).astype(x.dtype)
